# baseline (speedup 1.0000x reference)
import functools

import jax
import jax.numpy as jnp
from jax import lax
from jax.experimental import pallas as pl
from jax.experimental.pallas import tpu as pltpu
from jax.experimental.pallas import tpu_sc as plsc

B = 16384
F = 64
H = 100
NC = 2
NS = 16
NW = NC * NS
ROWS_W = B // NW
CHUNK = 128
NCH = ROWS_W // CHUNK


def _sc_gather(uf2, ub1, if2, ib1, upair, ipair, uidx, iidx):
  mesh = plsc.VectorSubcoreMesh(core_axis_name="c", subcore_axis_name="s")

  @functools.partial(
      pl.kernel,
      mesh=mesh,
      out_type=[
          jax.ShapeDtypeStruct((B, 128), jnp.float32),
          jax.ShapeDtypeStruct((B,), jnp.float32),
          jax.ShapeDtypeStruct((B, 128), jnp.float32),
          jax.ShapeDtypeStruct((B,), jnp.float32),
      ],
      scratch_types=[
          pltpu.VMEM((NCH, CHUNK), jnp.int32),
          pltpu.VMEM((NCH, CHUNK), jnp.int32),
          pltpu.VMEM((NCH, CHUNK), jnp.int32),
          pltpu.VMEM((NCH, CHUNK), jnp.int32),
          pltpu.VMEM((CHUNK, 128), jnp.float32),
          pltpu.VMEM((CHUNK, 128), jnp.float32),
          pltpu.VMEM((CHUNK, 128), jnp.float32),
          pltpu.VMEM((CHUNK, 128), jnp.float32),
          pltpu.VMEM((ROWS_W,), jnp.float32),
          pltpu.VMEM((ROWS_W,), jnp.float32),
          pltpu.SemaphoreType.DMA,
          pltpu.SemaphoreType.DMA,
          pltpu.SemaphoreType.DMA,
      ],
  )
  def k(uf_h, ubt_h, itf_h, ibt_h, up_h, ip_h, ui_h, ii_h,
        u_out, ub_out, it_out, ib_out,
        up_v, ip_v, ui_v, ii_v, u_a, u_b, it_a, it_b, ub_v, ib_v,
        sem_a, sem_b, sem_c):
    wid = lax.axis_index("s") * NC + lax.axis_index("c")
    base = wid * ROWS_W
    pltpu.sync_copy(up_h.at[wid], up_v)
    pltpu.sync_copy(ip_h.at[wid], ip_v)
    pltpu.sync_copy(ui_h.at[wid], ui_v)
    pltpu.sync_copy(ii_h.at[wid], ii_v)
    bias_copies = []
    for j in range(NCH):
      r = pl.ds(j * CHUNK, CHUNK)
      bias_copies.append(
          pltpu.async_copy(ubt_h.at[ui_v.at[j]], ub_v.at[r], sem_c))
      bias_copies.append(
          pltpu.async_copy(ibt_h.at[ii_v.at[j]], ib_v.at[r], sem_c))
    bufs = [(u_a, it_a, sem_a), (u_b, it_b, sem_b)]

    def issue(j):
      u_buf, it_buf, sem = bufs[j % 2]
      return (pltpu.async_copy(uf_h.at[up_v.at[j]], u_buf, sem),
              pltpu.async_copy(itf_h.at[ip_v.at[j]], it_buf, sem))

    cur = issue(0)
    for j in range(NCH):
      nxt = issue(j + 1) if j + 1 < NCH else None
      for c in cur:
        c.wait()
      u_buf, it_buf, _ = bufs[j % 2]
      s = pl.ds(base + j * CHUNK, CHUNK)
      pltpu.sync_copy(u_buf, u_out.at[s])
      pltpu.sync_copy(it_buf, it_out.at[s])
      cur = nxt
    for c in bias_copies:
      c.wait()
    s = pl.ds(base, ROWS_W)
    pltpu.sync_copy(ub_v, ub_out.at[s])
    pltpu.sync_copy(ib_v, ib_out.at[s])

  return k(uf2, ub1, if2, ib1, upair, ipair, uidx, iidx)


BLK = 2048


def _tc_body(u128_r, ub_r, it128_r, ib_r, ui_r, ii_r,
             w1u_r, w1ub_r, w1i_r, w1ib_r, b1_r,
             w2_r, b2_r, w3_r, b3_r, w4_r, b4_r, sd_r, out_r):
  u128 = u128_r[...]
  it128 = it128_r[...]
  uodd = lax.bitwise_and(ui_r[...], 1) == 1
  iodd = lax.bitwise_and(ii_r[...], 1) == 1
  u = jnp.where(uodd, u128[:, F:2 * F], u128[:, 0:F])
  it = jnp.where(iodd, it128[:, F:2 * F], it128[:, 0:F])
  ub = ub_r[...]
  ib = ib_r[...]
  sd = jnp.sum(u * it, axis=1, keepdims=True) + ub + ib
  h = jnp.dot(u, w1u_r[...], preferred_element_type=jnp.float32)
  h = h + jnp.dot(it, w1i_r[...], preferred_element_type=jnp.float32)
  h = h + ub * w1ub_r[...] + ib * w1ib_r[...] + b1_r[...]
  h = jnp.maximum(h, 0.0)
  h = jnp.dot(h, w2_r[...], preferred_element_type=jnp.float32) + b2_r[...]
  h = jnp.maximum(h, 0.0)
  h = jnp.dot(h, w3_r[...], preferred_element_type=jnp.float32) + b3_r[...] + sd
  out = jnp.dot(h, w4_r[...], preferred_element_type=jnp.float32) + b4_r[...]
  sd_r[...] = sd
  out_r[...] = out


def _tc_mlp(u128, ub, it128, ib, ui, ii,
            w1u, w1ub, w1i, w1ib, b1, w2, b2, w3, b3, w4, b4):
  full = lambda shape: pl.BlockSpec(shape, lambda i: (0, 0))
  rows = lambda shape: pl.BlockSpec(shape, lambda i: (i, 0))
  return pl.pallas_call(
      _tc_body,
      grid=(B // BLK,),
      in_specs=[
          rows((BLK, 128)), rows((BLK, 1)), rows((BLK, 128)), rows((BLK, 1)),
          rows((BLK, 1)), rows((BLK, 1)),
          full((F, H)), full((1, H)), full((F, H)), full((1, H)), full((1, H)),
          full((H, H)), full((1, H)), full((H, H)), full((1, H)),
          full((H, 1)), full((1, 1)),
      ],
      out_specs=[rows((BLK, 1)), rows((BLK, 1))],
      out_shape=[
          jax.ShapeDtypeStruct((B, 1), jnp.float32),
          jax.ShapeDtypeStruct((B, 1), jnp.float32),
      ],
  )(u128, ub, it128, ib, ui, ii,
    w1u, w1ub, w1i, w1ib, b1, w2, b2, w3, b3, w4, b4)


def kernel(item_in, user_in, user_factors, user_bias, item_factors, item_bias,
           W1, b1, W2, b2, W3, b3, W4, b4):
  uidx = user_in.reshape(NW, NCH, CHUNK)
  iidx = item_in.reshape(NW, NCH, CHUNK)
  upair = lax.shift_right_logical(uidx, 1)
  ipair = lax.shift_right_logical(iidx, 1)
  uf2 = user_factors.reshape(F * 100000 // 128, 128)
  if2 = item_factors.reshape(F * 100000 // 128, 128)
  ub1 = user_bias.reshape(-1)
  ib1 = item_bias.reshape(-1)
  u128, ubg, it128, ibg = _sc_gather(uf2, ub1, if2, ib1,
                                     upair, ipair, uidx, iidx)
  w1u = W1[0:F]
  w1ub = W1[F:F + 1]
  w1i = W1[F + 1:2 * F + 1]
  w1ib = W1[2 * F + 1:2 * F + 2]
  sd, out = _tc_mlp(u128, ubg.reshape(B, 1), it128, ibg.reshape(B, 1),
                    user_in, item_in,
                    w1u, w1ub, w1i, w1ib, b1.reshape(1, H),
                    W2, b2.reshape(1, H), W3, b3.reshape(1, H), W4,
                    b4.reshape(1, 1))
  return (sd, out)


# speedup vs baseline: 1.0524x; 1.0524x over previous
import functools

import jax
import jax.numpy as jnp
from jax import lax
from jax.experimental import pallas as pl
from jax.experimental.pallas import tpu as pltpu
from jax.experimental.pallas import tpu_sc as plsc

B = 16384
F = 64
H = 100
NC = 2
NS = 16
NW = NC * NS
ROWS_W = B // NW
CHUNK = 128
NCH = ROWS_W // CHUNK


def _sc_gather(uf, ub1, itf, ib1, uidx, iidx):
  mesh = plsc.VectorSubcoreMesh(core_axis_name="c", subcore_axis_name="s")

  @functools.partial(
      pl.kernel,
      mesh=mesh,
      out_type=[
          jax.ShapeDtypeStruct((B, 2 * F), jnp.float32),
          jax.ShapeDtypeStruct((B,), jnp.float32),
          jax.ShapeDtypeStruct((B,), jnp.float32),
      ],
      scratch_types=[
          pltpu.VMEM((NCH, CHUNK), jnp.int32),
          pltpu.VMEM((NCH, CHUNK), jnp.int32),
          pltpu.VMEM((CHUNK, F), jnp.float32),
          pltpu.VMEM((CHUNK, F), jnp.float32),
          pltpu.VMEM((CHUNK, F), jnp.float32),
          pltpu.VMEM((CHUNK, F), jnp.float32),
          pltpu.VMEM((ROWS_W,), jnp.float32),
          pltpu.VMEM((ROWS_W,), jnp.float32),
          pltpu.SemaphoreType.DMA,
          pltpu.SemaphoreType.DMA,
          pltpu.SemaphoreType.DMA,
      ],
      compiler_params=pltpu.CompilerParams(use_tc_tiling_on_sc=False),
  )
  def k(uf_h, ubt_h, itf_h, ibt_h, ui_h, ii_h,
        x_out, ub_out, ib_out,
        ui_v, ii_v, u_a, u_b, it_a, it_b, ub_v, ib_v,
        sem_a, sem_b, sem_c):
    wid = lax.axis_index("s") * NC + lax.axis_index("c")
    base = wid * ROWS_W
    pltpu.sync_copy(ui_h.at[wid], ui_v)
    pltpu.sync_copy(ii_h.at[wid], ii_v)
    bias_copies = []
    for j in range(NCH):
      r = pl.ds(j * CHUNK, CHUNK)
      bias_copies.append(
          pltpu.async_copy(ubt_h.at[ui_v.at[j]], ub_v.at[r], sem_c))
      bias_copies.append(
          pltpu.async_copy(ibt_h.at[ii_v.at[j]], ib_v.at[r], sem_c))
    bufs = [(u_a, it_a, sem_a), (u_b, it_b, sem_b)]

    def issue(j):
      u_buf, it_buf, sem = bufs[j % 2]
      return (pltpu.async_copy(uf_h.at[ui_v.at[j]], u_buf, sem),
              pltpu.async_copy(itf_h.at[ii_v.at[j]], it_buf, sem))

    cur = issue(0)
    for j in range(NCH):
      nxt = issue(j + 1) if j + 1 < NCH else None
      for c in cur:
        c.wait()
      u_buf, it_buf, _ = bufs[j % 2]
      rows = pl.ds(base + j * CHUNK, CHUNK)
      pltpu.sync_copy(u_buf, x_out.at[rows, pl.ds(0, F)])
      pltpu.sync_copy(it_buf, x_out.at[rows, pl.ds(F, F)])
      cur = nxt
    for c in bias_copies:
      c.wait()
    s = pl.ds(base, ROWS_W)
    pltpu.sync_copy(ub_v, ub_out.at[s])
    pltpu.sync_copy(ib_v, ib_out.at[s])

  return k(uf, ub1, itf, ib1, uidx, iidx)


BLK = 2048


def _tc_body(x_r, ub_r, ib_r,
             w1u_r, w1ub_r, w1i_r, w1ib_r, b1_r,
             w2_r, b2_r, w3_r, b3_r, w4_r, b4_r, sd_r, out_r):
  x = x_r[...]
  u = x[:, 0:F]
  it = x[:, F:2 * F]
  ub = ub_r[...]
  ib = ib_r[...]
  sd = jnp.sum(u * it, axis=1, keepdims=True) + ub + ib
  h = jnp.dot(u, w1u_r[...], preferred_element_type=jnp.float32)
  h = h + jnp.dot(it, w1i_r[...], preferred_element_type=jnp.float32)
  h = h + ub * w1ub_r[...] + ib * w1ib_r[...] + b1_r[...]
  h = jnp.maximum(h, 0.0)
  h = jnp.dot(h, w2_r[...], preferred_element_type=jnp.float32) + b2_r[...]
  h = jnp.maximum(h, 0.0)
  h = jnp.dot(h, w3_r[...], preferred_element_type=jnp.float32) + b3_r[...] + sd
  out = jnp.dot(h, w4_r[...], preferred_element_type=jnp.float32) + b4_r[...]
  sd_r[...] = sd
  out_r[...] = out


def _tc_mlp(x, ub, ib, w1u, w1ub, w1i, w1ib, b1, w2, b2, w3, b3, w4, b4):
  full = lambda shape: pl.BlockSpec(shape, lambda i: (0, 0))
  rows = lambda shape: pl.BlockSpec(shape, lambda i: (i, 0))
  return pl.pallas_call(
      _tc_body,
      grid=(B // BLK,),
      in_specs=[
          rows((BLK, 2 * F)), rows((BLK, 1)), rows((BLK, 1)),
          full((F, H)), full((1, H)), full((F, H)), full((1, H)), full((1, H)),
          full((H, H)), full((1, H)), full((H, H)), full((1, H)),
          full((H, 1)), full((1, 1)),
      ],
      out_specs=[rows((BLK, 1)), rows((BLK, 1))],
      out_shape=[
          jax.ShapeDtypeStruct((B, 1), jnp.float32),
          jax.ShapeDtypeStruct((B, 1), jnp.float32),
      ],
  )(x, ub, ib, w1u, w1ub, w1i, w1ib, b1, w2, b2, w3, b3, w4, b4)


def kernel(item_in, user_in, user_factors, user_bias, item_factors, item_bias,
           W1, b1, W2, b2, W3, b3, W4, b4):
  uidx = user_in.reshape(NW, NCH, CHUNK)
  iidx = item_in.reshape(NW, NCH, CHUNK)
  ub1 = user_bias.reshape(-1)
  ib1 = item_bias.reshape(-1)
  ufT, ifT = jax.lax.optimization_barrier((user_factors.T, item_factors.T))
  x, ubg, ibg = _sc_gather(ufT.T, ub1, ifT.T, ib1, uidx, iidx)
  w1u = W1[0:F]
  w1ub = W1[F:F + 1]
  w1i = W1[F + 1:2 * F + 1]
  w1ib = W1[2 * F + 1:2 * F + 2]
  sd, out = _tc_mlp(x, ubg.reshape(B, 1), ibg.reshape(B, 1),
                    w1u, w1ub, w1i, w1ib, b1.reshape(1, H),
                    W2, b2.reshape(1, H), W3, b3.reshape(1, H), W4,
                    b4.reshape(1, 1))
  return (sd, out)
